# Initial kernel scaffold; baseline (speedup 1.0000x reference)
import jax
import jax.numpy as jnp
from jax.experimental import pallas as pl


def kernel(x, W1_l, b1_l, W1_r, W2_l, b2_l, W2_r):
    raise NotImplementedError("write your pallas kernel here")


# speedup vs baseline: 1045.6180x; 1045.6180x over previous
import jax
import jax.numpy as jnp
from jax.experimental import pallas as pl

N = 512


def _fused_gnn(x_ref, w1l_ref, b1l_ref, w1r_ref, w2l_ref, b2l_ref, w2r_ref,
               out_ref):
    inv = 1.0 / (N - 1)
    x = x_ref[...]

    s_x = jnp.sum(x, axis=0, keepdims=True)
    a1 = w1r_ref[...] - w1l_ref[...] * inv
    c1 = jax.lax.dot_general(
        s_x, w1l_ref[...], (((1,), (1,)), ((), ())),
        preferred_element_type=jnp.float32) * inv + b1l_ref[...]
    h = jax.lax.dot_general(
        x, a1, (((1,), (1,)), ((), ())),
        preferred_element_type=jnp.float32) + c1
    h = jnp.maximum(h, 0.0)

    s_h = jnp.sum(h, axis=0, keepdims=True)
    a2 = w2r_ref[...] - w2l_ref[...] * inv
    c2 = jax.lax.dot_general(
        s_h, w2l_ref[...], (((1,), (1,)), ((), ())),
        preferred_element_type=jnp.float32) * inv + b2l_ref[...]
    out_ref[...] = jax.lax.dot_general(
        h, a2, (((1,), (1,)), ((), ())),
        preferred_element_type=jnp.float32) + c2


def kernel(x, W1_l, b1_l, W1_r, W2_l, b2_l, W2_r):
    D = x.shape[1]
    return pl.pallas_call(
        _fused_gnn,
        out_shape=jax.ShapeDtypeStruct((N, D), jnp.float32),
    )(x, W1_l, b1_l.reshape(1, -1), W1_r, W2_l, b2_l.reshape(1, -1), W2_r)
